# baseline (speedup 1.0000x reference)
import functools

import jax
import jax.numpy as jnp
from jax import lax
from jax.experimental import pallas as pl
from jax.experimental.pallas import tpu as pltpu
from jax.experimental.pallas import tpu_sc as plsc

E = 8
K = 2
H = 1024
T = 2048
CAP = 640
AUX_COEF = 0.01
Z_COEF = 0.001
BT = 256
G = T // BT

NC = 2
NS = 16
NW = NC * NS
TPW = T // NW
R = 8
NG = TPW // R
ROWW = E * CAP


def _gate_kernel(x_ref, w_ref, dispatch_ref, idx_ref, scol_ref,
                 sval_ref, lb_ref, z_ref, counts_ref, psum_ref, zsum_ref):
    i = pl.program_id(0)

    @pl.when(i == 0)
    def _init():
        counts_ref[...] = jnp.zeros_like(counts_ref)
        psum_ref[...] = jnp.zeros_like(psum_ref)
        zsum_ref[...] = jnp.zeros_like(zsum_ref)

    x = x_ref[...]
    w = w_ref[...]
    logits = jax.lax.dot_general(
        x, w, (((1,), (1,)), ((), ())),
        preferred_element_type=jnp.float32)

    col = jax.lax.broadcasted_iota(jnp.int32, (BT, E), 1)
    m0 = jnp.max(logits, axis=1, keepdims=True)
    i0 = jnp.min(jnp.where(logits == m0, col, E), axis=1, keepdims=True)
    masked = jnp.where(col == i0, -jnp.inf, logits)
    m1 = jnp.max(masked, axis=1, keepdims=True)
    i1 = jnp.min(jnp.where(masked == m1, col, E), axis=1, keepdims=True)

    t = jnp.exp(m1 - m0)
    w0 = 1.0 / (1.0 + t)
    w1 = t / (1.0 + t)

    ex = jnp.exp(logits - m0)
    zdenom = jnp.sum(ex, axis=1, keepdims=True)
    probs = ex / zdenom
    psum_ref[...] += jnp.sum(probs, axis=0, keepdims=True)
    zsum_ref[...] += jnp.sum(m0 + jnp.log(zdenom)).reshape(1, 1)

    a = (col == i0).astype(jnp.float32) + (col == i1).astype(jnp.float32)

    r_i = jax.lax.broadcasted_iota(jnp.int32, (BT, BT), 0)
    c_i = jax.lax.broadcasted_iota(jnp.int32, (BT, BT), 1)
    tri = (r_i > c_i).astype(jnp.float32)
    c_local = jax.lax.dot_general(
        tri, a, (((1,), (0,)), ((), ())),
        preferred_element_type=jnp.float32)
    c_global = c_local + counts_ref[...]

    p0 = jnp.sum(jnp.where(col == i0, c_global, 0.0), axis=1, keepdims=True)
    p1 = jnp.sum(jnp.where(col == i1, c_global, 0.0), axis=1, keepdims=True)
    p0 = p0.astype(jnp.int32)
    p1 = p1.astype(jnp.int32)

    counts_ref[...] += jnp.sum(a, axis=0, keepdims=True)

    q0 = jnp.where(p0 < CAP, i0 * CAP + p0, -1)
    q1 = jnp.where(p1 < CAP, i1 * CAP + p1, -1)
    cq = jax.lax.broadcasted_iota(jnp.int32, (BT, E * CAP), 1)
    d2 = (cq == q0) | (cq == q1)
    for e in range(E):
        dispatch_ref[:, e, :] = d2[:, e * CAP:(e + 1) * CAP]

    idx_ref[...] = jnp.concatenate([i0, i1], axis=1)

    sc0 = i0 * CAP + jnp.minimum(p0, CAP - 1)
    sc1 = i1 * CAP + jnp.minimum(p1, CAP - 1)
    scol_ref[...] = jnp.concatenate([sc0, sc1], axis=1)
    sval_ref[...] = jnp.concatenate(
        [jnp.where(p0 < CAP, w0, 0.0), jnp.where(p1 < CAP, w1, 0.0)], axis=1)

    tpe = jnp.minimum(counts_ref[...], float(CAP))
    tpe = tpe / jnp.sum(tpe)
    mean_prob = psum_ref[...] / float(T)
    lb_ref[...] = (AUX_COEF * E * jnp.sum(mean_prob * tpe)).reshape(1, 1)
    z_ref[...] = (Z_COEF * zsum_ref[...] / float(T)).reshape(1, 1)


def _tc_gate(x, W_gate):
    return pl.pallas_call(
        _gate_kernel,
        grid=(G,),
        in_specs=[
            pl.BlockSpec((BT, H), lambda i: (i, 0)),
            pl.BlockSpec((E, H), lambda i: (0, 0)),
        ],
        out_specs=[
            pl.BlockSpec((BT, E, CAP), lambda i: (i, 0, 0)),
            pl.BlockSpec((BT, K), lambda i: (i, 0)),
            pl.BlockSpec((BT, K), lambda i: (i, 0)),
            pl.BlockSpec((BT, K), lambda i: (i, 0)),
            pl.BlockSpec((1, 1), lambda i: (0, 0)),
            pl.BlockSpec((1, 1), lambda i: (0, 0)),
        ],
        out_shape=[
            jax.ShapeDtypeStruct((T, E, CAP), jnp.bool_),
            jax.ShapeDtypeStruct((T, K), jnp.int32),
            jax.ShapeDtypeStruct((T, K), jnp.int32),
            jax.ShapeDtypeStruct((T, K), jnp.float32),
            jax.ShapeDtypeStruct((1, 1), jnp.float32),
            jax.ShapeDtypeStruct((1, 1), jnp.float32),
        ],
        scratch_shapes=[
            pltpu.VMEM((1, E), jnp.float32),
            pltpu.VMEM((1, E), jnp.float32),
            pltpu.VMEM((1, 1), jnp.float32),
        ],
        compiler_params=pltpu.CompilerParams(
            dimension_semantics=("arbitrary",),
        ),
    )(x, W_gate)


def _combine_body(scol, sval, out, colv, valv, buf0, buf1, sem0, sem1):
    wid = lax.axis_index("s") * NC + lax.axis_index("c")
    base = wid * TPW

    pltpu.sync_copy(scol.at[pl.ds(base * K, TPW * K)], colv)
    pltpu.sync_copy(sval.at[pl.ds(base * K, TPW * K)], valv)

    z16 = jnp.zeros((16,), jnp.float32)
    bufs = (buf0, buf1)
    sems = (sem0, sem1)

    for buf in bufs:
        def _zero_chunk(j, carry, buf=buf):
            for l in range(16):
                buf[pl.ds(j * 256 + l * 16, 16)] = z16
            return carry
        lax.fori_loop(0, R * ROWW // 256, _zero_chunk, 0)

    tg = lax.iota(jnp.int32, 16) >> 1

    def _idx(g):
        colg = colv[pl.ds(g * 16, 16)]
        return tg * ROWW + colg

    descs = [None, None]
    for g in range(NG):
        b = g & 1
        if descs[b] is not None:
            descs[b].wait()
            plsc.store_scatter(bufs[b], [_idx(g - 2)], z16)
        valg = valv[pl.ds(g * 16, 16)]
        plsc.store_scatter(bufs[b], [_idx(g)], valg)
        descs[b] = pltpu.async_copy(
            bufs[b], out.at[pl.ds((base + g * R) * ROWW, R * ROWW)], sems[b])
    for b in (0, 1):
        descs[b].wait()


def _make_combine_sc():
    return pl.kernel(
        _combine_body,
        out_type=jax.ShapeDtypeStruct((T * E * CAP,), jnp.float32),
        mesh=plsc.VectorSubcoreMesh(core_axis_name="c", subcore_axis_name="s",
                                    num_cores=NC, num_subcores=NS),
        scratch_types=[
            pltpu.VMEM((TPW * K,), jnp.int32),
            pltpu.VMEM((TPW * K,), jnp.float32),
            pltpu.VMEM((R * ROWW,), jnp.float32),
            pltpu.VMEM((R * ROWW,), jnp.float32),
            pltpu.SemaphoreType.DMA,
            pltpu.SemaphoreType.DMA,
        ],
        compiler_params=pltpu.CompilerParams(needs_layout_passes=False),
    )


@jax.jit
def kernel(hidden_states, W_gate):
    x = hidden_states.reshape(T, H)
    dispatch, idx, scol, sval, lb, z = _tc_gate(x, W_gate)
    combine_flat = _make_combine_sc()(
        scol.reshape(T * K), sval.reshape(T * K))
    combine = combine_flat.reshape(T, E, CAP)
    return dispatch, combine, idx, lb.reshape(()), z.reshape(())
